# baseline (speedup 1.0000x reference)
import functools

import jax
import jax.numpy as jnp
from jax import lax
from jax.experimental import pallas as pl
from jax.experimental.pallas import tpu as pltpu


def _compute_one(x, wd1, wd2, bd1, bd2, b1_ref, wse1a_ref, wse1b_ref,
                 wse2a_ref, wse2b_ref, w2a_ref, w2b_ref, b2_ref, w1, H):
    inC, Mo = x.shape
    exp = w1.shape[1]
    W = Mo // H
    f32 = jnp.float32

    y = lax.dot_general(x.astype(jnp.bfloat16), w1,
                        (((0,), (0,)), ((), ())),
                        preferred_element_type=f32)
    y = y + b1_ref[...]
    y = y * jnp.clip(y + 3.0, 0.0, 6.0)
    y3 = y.reshape(H, W, exp)

    S = jnp.sum(y, axis=0, keepdims=True)
    cs0 = jnp.sum(y3[:, 0, :], axis=0, keepdims=True)
    csW = jnp.sum(y3[:, W - 1, :], axis=0, keepdims=True)
    rs0 = jnp.sum(y3[0], axis=0, keepdims=True)
    rsH = jnp.sum(y3[H - 1], axis=0, keepdims=True)
    inv = 1.0 / float(Mo)
    p1 = (wd1[0:1] * (S - csW) + wd1[1:2] * S + wd1[2:3] * (S - cs0)) * inv + bd1
    p2 = (wd2[0:1] * (S - rsH) + wd2[1:2] * S + wd2[2:3] * (S - rs0)) * inv + bd2

    h = (jnp.dot(p1, wse1a_ref[...], preferred_element_type=f32)
         + jnp.dot(p2, wse1b_ref[...], preferred_element_type=f32))
    h = jnp.maximum(h, 0.0)
    se1 = jnp.clip(jnp.dot(h, wse2a_ref[...], preferred_element_type=f32)
                   + 3.0, 0.0, 6.0) * (1.0 / 6.0)
    se2 = jnp.clip(jnp.dot(h, wse2b_ref[...], preferred_element_type=f32)
                   + 3.0, 0.0, 6.0) * (1.0 / 6.0)
    wd1s = wd1 * se1
    wd2s = wd2 * se2
    bd1s = (bd1 * se1).reshape(1, 1, exp)
    bd2s = (bd2 * se2).reshape(1, 1, exp)

    zc = jnp.zeros((H, 1, exp), f32)
    u1 = (wd1s[0].reshape(1, 1, exp) * jnp.concatenate([zc, y3[:, :W - 1]], 1)
          + wd1s[1].reshape(1, 1, exp) * y3
          + wd1s[2].reshape(1, 1, exp) * jnp.concatenate([y3[:, 1:], zc], 1)
          + bd1s)
    x1 = (u1 * jnp.clip(u1 + 3.0, 0.0, 6.0)).reshape(Mo, exp).astype(jnp.bfloat16)

    zr = jnp.zeros((1, W, exp), f32)
    u2 = (wd2s[0].reshape(1, 1, exp) * jnp.concatenate([zr, y3[:H - 1]], 0)
          + wd2s[1].reshape(1, 1, exp) * y3
          + wd2s[2].reshape(1, 1, exp) * jnp.concatenate([y3[1:], zr], 0)
          + bd2s)
    x2 = (u2 * jnp.clip(u2 + 3.0, 0.0, 6.0)).reshape(Mo, exp).astype(jnp.bfloat16)

    dn = (((1,), (1,)), ((), ()))
    out = lax.dot_general(w2a_ref[...], x1, dn, preferred_element_type=f32)
    out = out + lax.dot_general(w2b_ref[...], x2, dn, preferred_element_type=f32)
    return out + b2_ref[...]


def _block_kernel(x_hbm, w1_ref, b1_ref, wd1_ref, bd1_ref, wd2_ref,
                  bd2_ref, wse1a_ref, wse1b_ref, wse2a_ref, wse2b_ref,
                  w2a_ref, w2b_ref, b2_ref, o_hbm,
                  x_buf, o_buf, in_sem, out_sem, *, H):
    B = x_hbm.shape[0]
    w1 = w1_ref[...]
    wd1 = wd1_ref[...]
    wd2 = wd2_ref[...]
    bd1 = bd1_ref[...]
    bd2 = bd2_ref[...]

    def dma_in(slot, step):
        pltpu.make_async_copy(x_hbm.at[step], x_buf.at[slot],
                              in_sem.at[slot]).start()

    def wait_in(slot):
        pltpu.make_async_copy(x_hbm.at[0], x_buf.at[slot],
                              in_sem.at[slot]).wait()

    def dma_out(slot, step):
        pltpu.make_async_copy(o_buf.at[slot], o_hbm.at[step],
                              out_sem.at[slot]).start()

    def wait_out(slot):
        pltpu.make_async_copy(o_buf.at[0], o_hbm.at[0],
                              out_sem.at[slot]).wait()

    dma_in(0, 0)

    def body(step, _):
        cur = lax.rem(step, 2)
        nxt = lax.rem(step + 1, 2)

        @pl.when(step + 1 < B)
        def _():
            dma_in(nxt, step + 1)

        wait_in(cur)

        @pl.when(step >= 2)
        def _():
            wait_out(cur)

        res = _compute_one(x_buf[cur], wd1, wd2, bd1, bd2, b1_ref,
                           wse1a_ref, wse1b_ref, wse2a_ref, wse2b_ref,
                           w2a_ref, w2b_ref, b2_ref, w1, H)
        o_buf[cur] = res.astype(jnp.bfloat16)
        dma_out(cur, step)
        return ()

    lax.fori_loop(0, B, body, (), unroll=False)
    wait_out(lax.rem(B - 2, 2))
    wait_out(lax.rem(B - 1, 2))


def kernel(x_nchw, w1, bn1_s, bn1_b, wd1, bnd1_s, bnd1_b, wd2, bnd2_s, bnd2_b,
           w_se1, w_se2, w2, bn2_s, bn2_b):
    f32, bf16 = jnp.float32, jnp.bfloat16
    B, inC, H, W = x_nchw.shape
    Mo = H * W
    exp = w1.shape[1]
    oup = w2.shape[1]

    w1f = (w1 * bn1_s).astype(bf16)
    b1 = bn1_b.astype(f32)
    wd1f = (wd1 * bnd1_s * (1.0 / 6.0)).astype(f32)
    wd2f = (wd2 * bnd2_s * (1.0 / 6.0)).astype(f32)
    bd1 = bnd1_b.astype(f32)
    bd2 = bnd2_b.astype(f32)
    w2f = w2 * bn2_s
    w2a = (jnp.transpose(w2f[:exp]) * (1.0 / 6.0)).astype(bf16)
    w2b = (jnp.transpose(w2f[exp:]) * (1.0 / 6.0)).astype(bf16)
    b2 = bn2_b.reshape(oup, 1).astype(f32)
    wse1a = w_se1[:exp].astype(f32)
    wse1b = w_se1[exp:].astype(f32)
    wse2a = w_se2[:, :exp].astype(f32)
    wse2b = w_se2[:, exp:].astype(f32)

    x3 = x_nchw.reshape(B, inC, Mo)

    vspec = pl.BlockSpec(memory_space=pltpu.MemorySpace.VMEM)
    aspec = pl.BlockSpec(memory_space=pl.ANY)
    out = pl.pallas_call(
        functools.partial(_block_kernel, H=H),
        out_shape=jax.ShapeDtypeStruct((B, oup, Mo), bf16),
        in_specs=[aspec] + [vspec] * 13,
        out_specs=aspec,
        scratch_shapes=[
            pltpu.VMEM((2, inC, Mo), f32),
            pltpu.VMEM((2, oup, Mo), bf16),
            pltpu.SemaphoreType.DMA((2,)),
            pltpu.SemaphoreType.DMA((2,)),
        ],
        compiler_params=pltpu.CompilerParams(
            vmem_limit_bytes=64 * 1024 * 1024),
    )(x3, w1f, b1, wd1f, bd1, wd2f, bd2,
      wse1a, wse1b, wse2a, wse2b, w2a, w2b, b2)
    return out.reshape(B, oup, H, W).astype(f32)


# speedup vs baseline: 1.0360x; 1.0360x over previous
import functools

import jax
import jax.numpy as jnp
from jax import lax
from jax.experimental import pallas as pl
from jax.experimental.pallas import tpu as pltpu


def _block_kernel(x_ref, w1_ref, b1_ref, wd1_ref, bd1_ref, wd2_ref,
                  bd2_ref, wse1a_ref, wse1b_ref, wse2a_ref, wse2b_ref,
                  w2a_ref, w2b_ref, b2_ref, o_ref, *, H):
    nb, inC, Mo = x_ref.shape
    exp = w1_ref.shape[1]
    W = Mo // H
    f32 = jnp.float32

    for j in range(nb):
        _one_batch(x_ref, w1_ref, b1_ref, wd1_ref, bd1_ref, wd2_ref,
                   bd2_ref, wse1a_ref, wse1b_ref, wse2a_ref, wse2b_ref,
                   w2a_ref, w2b_ref, b2_ref, o_ref, j, H, W, Mo, exp, f32)


def _one_batch(x_ref, w1_ref, b1_ref, wd1_ref, bd1_ref, wd2_ref,
               bd2_ref, wse1a_ref, wse1b_ref, wse2a_ref, wse2b_ref,
               w2a_ref, w2b_ref, b2_ref, o_ref, j, H, W, Mo, exp, f32):
    y = lax.dot_general(x_ref[j].astype(jnp.bfloat16), w1_ref[...],
                        (((0,), (0,)), ((), ())),
                        preferred_element_type=f32)
    y = y + b1_ref[...]
    y = y * jnp.clip(y + 3.0, 0.0, 6.0)
    y3 = y.reshape(H, W, exp)

    wd1 = wd1_ref[...]
    wd2 = wd2_ref[...]
    bd1 = bd1_ref[...]
    bd2 = bd2_ref[...]

    S = jnp.sum(y, axis=0, keepdims=True)
    cs0 = jnp.sum(y3[:, 0, :], axis=0, keepdims=True)
    csW = jnp.sum(y3[:, W - 1, :], axis=0, keepdims=True)
    rs0 = jnp.sum(y3[0], axis=0, keepdims=True)
    rsH = jnp.sum(y3[H - 1], axis=0, keepdims=True)
    inv = 1.0 / float(Mo)
    p1 = (wd1[0:1] * (S - csW) + wd1[1:2] * S + wd1[2:3] * (S - cs0)) * inv + bd1
    p2 = (wd2[0:1] * (S - rsH) + wd2[1:2] * S + wd2[2:3] * (S - rs0)) * inv + bd2

    h = (jnp.dot(p1, wse1a_ref[...], preferred_element_type=f32)
         + jnp.dot(p2, wse1b_ref[...], preferred_element_type=f32))
    h = jnp.maximum(h, 0.0)
    se1 = jnp.clip(jnp.dot(h, wse2a_ref[...], preferred_element_type=f32)
                   + 3.0, 0.0, 6.0) * (1.0 / 6.0)
    se2 = jnp.clip(jnp.dot(h, wse2b_ref[...], preferred_element_type=f32)
                   + 3.0, 0.0, 6.0) * (1.0 / 6.0)
    wd1s = wd1 * se1
    wd2s = wd2 * se2
    bd1s = (bd1 * se1).reshape(1, 1, exp)
    bd2s = (bd2 * se2).reshape(1, 1, exp)

    zc = jnp.zeros((H, 1, exp), f32)
    u1 = (wd1s[0].reshape(1, 1, exp) * jnp.concatenate([zc, y3[:, :W - 1]], 1)
          + wd1s[1].reshape(1, 1, exp) * y3
          + wd1s[2].reshape(1, 1, exp) * jnp.concatenate([y3[:, 1:], zc], 1)
          + bd1s)
    x1 = (u1 * jnp.clip(u1 + 3.0, 0.0, 6.0)).reshape(Mo, exp).astype(jnp.bfloat16)

    zr = jnp.zeros((1, W, exp), f32)
    u2 = (wd2s[0].reshape(1, 1, exp) * jnp.concatenate([zr, y3[:H - 1]], 0)
          + wd2s[1].reshape(1, 1, exp) * y3
          + wd2s[2].reshape(1, 1, exp) * jnp.concatenate([y3[1:], zr], 0)
          + bd2s)
    x2 = (u2 * jnp.clip(u2 + 3.0, 0.0, 6.0)).reshape(Mo, exp).astype(jnp.bfloat16)

    dn = (((1,), (1,)), ((), ()))
    out = lax.dot_general(w2a_ref[...], x1, dn, preferred_element_type=f32)
    out = out + lax.dot_general(w2b_ref[...], x2, dn, preferred_element_type=f32)
    o_ref[j] = (out + b2_ref[...]).astype(jnp.bfloat16)


def kernel(x_nchw, w1, bn1_s, bn1_b, wd1, bnd1_s, bnd1_b, wd2, bnd2_s, bnd2_b,
           w_se1, w_se2, w2, bn2_s, bn2_b):
    f32, bf16 = jnp.float32, jnp.bfloat16
    B, inC, H, W = x_nchw.shape
    Mo = H * W
    exp = w1.shape[1]
    oup = w2.shape[1]

    w1f = (w1 * bn1_s).astype(bf16)
    b1 = bn1_b.astype(f32)
    wd1f = (wd1 * bnd1_s * (1.0 / 6.0)).astype(f32)
    wd2f = (wd2 * bnd2_s * (1.0 / 6.0)).astype(f32)
    bd1 = bnd1_b.astype(f32)
    bd2 = bnd2_b.astype(f32)
    w2f = w2 * bn2_s
    w2a = (jnp.transpose(w2f[:exp]) * (1.0 / 6.0)).astype(bf16)
    w2b = (jnp.transpose(w2f[exp:]) * (1.0 / 6.0)).astype(bf16)
    b2 = bn2_b.reshape(oup, 1).astype(f32)
    wse1a = w_se1[:exp].astype(f32)
    wse1b = w_se1[exp:].astype(f32)
    wse2a = w_se2[:, :exp].astype(f32)
    wse2b = w_se2[:, exp:].astype(f32)

    x3 = x_nchw.reshape(B, inC, Mo)

    const = lambda shape: pl.BlockSpec(shape, lambda b: tuple(0 for _ in shape))
    NB = 2
    out = pl.pallas_call(
        functools.partial(_block_kernel, H=H),
        out_shape=jax.ShapeDtypeStruct((B, oup, Mo), bf16),
        grid=(B // NB,),
        in_specs=[
            pl.BlockSpec((NB, inC, Mo), lambda b: (b, 0, 0)),
            const(w1f.shape), const(b1.shape),
            const(wd1f.shape), const(bd1.shape),
            const(wd2f.shape), const(bd2.shape),
            const(wse1a.shape), const(wse1b.shape),
            const(wse2a.shape), const(wse2b.shape),
            const(w2a.shape), const(w2b.shape), const(b2.shape),
        ],
        out_specs=pl.BlockSpec((NB, oup, Mo), lambda b: (b, 0, 0)),
        compiler_params=pltpu.CompilerParams(
            dimension_semantics=("arbitrary",),
            vmem_limit_bytes=100 * 1024 * 1024),
    )(x3, w1f, b1, wd1f, bd1, wd2f, bd2,
      wse1a, wse1b, wse2a, wse2b, w2a, w2b, b2)
    return out.reshape(B, oup, H, W).astype(f32)


# speedup vs baseline: 1.0368x; 1.0007x over previous
import functools

import jax
import jax.numpy as jnp
from jax import lax
from jax.experimental import pallas as pl
from jax.experimental.pallas import tpu as pltpu


def _block_kernel(x_ref, w1_ref, b1_ref, wd1_ref, bd1_ref, wd2_ref,
                  bd2_ref, wse1a_ref, wse1b_ref, wse2a_ref, wse2b_ref,
                  w2a_ref, w2b_ref, b2_ref, o_ref, *, H):
    nb, inC, Mo = x_ref.shape
    exp = w1_ref.shape[1]
    W = Mo // H
    f32 = jnp.float32

    for j in range(nb):
        _one_batch(x_ref, w1_ref, b1_ref, wd1_ref, bd1_ref, wd2_ref,
                   bd2_ref, wse1a_ref, wse1b_ref, wse2a_ref, wse2b_ref,
                   w2a_ref, w2b_ref, b2_ref, o_ref, j, H, W, Mo, exp, f32)


def _one_batch(x_ref, w1_ref, b1_ref, wd1_ref, bd1_ref, wd2_ref,
               bd2_ref, wse1a_ref, wse1b_ref, wse2a_ref, wse2b_ref,
               w2a_ref, w2b_ref, b2_ref, o_ref, j, H, W, Mo, exp, f32):
    y = lax.dot_general(x_ref[j].astype(jnp.bfloat16), w1_ref[...],
                        (((0,), (0,)), ((), ())),
                        preferred_element_type=f32)
    y = y + b1_ref[...]
    y = y * jnp.clip(y + 3.0, 0.0, 6.0)
    y3 = y.reshape(H, W, exp)

    wd1 = wd1_ref[...]
    wd2 = wd2_ref[...]
    bd1 = bd1_ref[...]
    bd2 = bd2_ref[...]

    S = jnp.sum(y, axis=0, keepdims=True)
    cs0 = jnp.sum(y3[:, 0, :], axis=0, keepdims=True)
    csW = jnp.sum(y3[:, W - 1, :], axis=0, keepdims=True)
    rs0 = jnp.sum(y3[0], axis=0, keepdims=True)
    rsH = jnp.sum(y3[H - 1], axis=0, keepdims=True)
    inv = 1.0 / float(Mo)
    p1 = (wd1[0:1] * (S - csW) + wd1[1:2] * S + wd1[2:3] * (S - cs0)) * inv + bd1
    p2 = (wd2[0:1] * (S - rsH) + wd2[1:2] * S + wd2[2:3] * (S - rs0)) * inv + bd2

    h = (jnp.dot(p1, wse1a_ref[...], preferred_element_type=f32)
         + jnp.dot(p2, wse1b_ref[...], preferred_element_type=f32))
    h = jnp.maximum(h, 0.0)
    se1 = jnp.clip(jnp.dot(h, wse2a_ref[...], preferred_element_type=f32)
                   + 3.0, 0.0, 6.0) * (1.0 / 6.0)
    se2 = jnp.clip(jnp.dot(h, wse2b_ref[...], preferred_element_type=f32)
                   + 3.0, 0.0, 6.0) * (1.0 / 6.0)
    wd1s = wd1 * se1
    wd2s = wd2 * se2
    bd1s = (bd1 * se1).reshape(1, 1, exp)
    bd2s = (bd2 * se2).reshape(1, 1, exp)

    zc = jnp.zeros((H, 1, exp), f32)
    u1 = (wd1s[0].reshape(1, 1, exp) * jnp.concatenate([zc, y3[:, :W - 1]], 1)
          + wd1s[1].reshape(1, 1, exp) * y3
          + wd1s[2].reshape(1, 1, exp) * jnp.concatenate([y3[:, 1:], zc], 1)
          + bd1s)
    x1 = (u1 * jnp.clip(u1 + 3.0, 0.0, 6.0)).reshape(Mo, exp).astype(jnp.bfloat16)

    zr = jnp.zeros((1, W, exp), f32)
    u2 = (wd2s[0].reshape(1, 1, exp) * jnp.concatenate([zr, y3[:H - 1]], 0)
          + wd2s[1].reshape(1, 1, exp) * y3
          + wd2s[2].reshape(1, 1, exp) * jnp.concatenate([y3[1:], zr], 0)
          + bd2s)
    x2 = (u2 * jnp.clip(u2 + 3.0, 0.0, 6.0)).reshape(Mo, exp).astype(jnp.bfloat16)

    dn = (((1,), (1,)), ((), ()))
    out = lax.dot_general(w2a_ref[...], x1, dn, preferred_element_type=f32)
    out = out + lax.dot_general(w2b_ref[...], x2, dn, preferred_element_type=f32)
    o_ref[j] = (out + b2_ref[...]).astype(jnp.bfloat16)


def kernel(x_nchw, w1, bn1_s, bn1_b, wd1, bnd1_s, bnd1_b, wd2, bnd2_s, bnd2_b,
           w_se1, w_se2, w2, bn2_s, bn2_b):
    f32, bf16 = jnp.float32, jnp.bfloat16
    B, inC, H, W = x_nchw.shape
    Mo = H * W
    exp = w1.shape[1]
    oup = w2.shape[1]

    w1f = (w1 * bn1_s).astype(bf16)
    b1 = bn1_b.astype(f32)
    wd1f = (wd1 * bnd1_s * (1.0 / 6.0)).astype(f32)
    wd2f = (wd2 * bnd2_s * (1.0 / 6.0)).astype(f32)
    bd1 = bnd1_b.astype(f32)
    bd2 = bnd2_b.astype(f32)
    w2f = w2 * bn2_s
    w2a = (jnp.transpose(w2f[:exp]) * (1.0 / 6.0)).astype(bf16)
    w2b = (jnp.transpose(w2f[exp:]) * (1.0 / 6.0)).astype(bf16)
    b2 = bn2_b.reshape(oup, 1).astype(f32)
    wse1a = w_se1[:exp].astype(f32)
    wse1b = w_se1[exp:].astype(f32)
    wse2a = w_se2[:, :exp].astype(f32)
    wse2b = w_se2[:, exp:].astype(f32)

    x3 = x_nchw.reshape(B, inC, Mo)

    const = lambda shape: pl.BlockSpec(shape, lambda b: tuple(0 for _ in shape))
    NB = 2
    out = pl.pallas_call(
        functools.partial(_block_kernel, H=H),
        out_shape=jax.ShapeDtypeStruct((B, oup, Mo), bf16),
        grid=(B // NB,),
        in_specs=[
            pl.BlockSpec((NB, inC, Mo), lambda b: (b, 0, 0)),
            const(w1f.shape), const(b1.shape),
            const(wd1f.shape), const(bd1.shape),
            const(wd2f.shape), const(bd2.shape),
            const(wse1a.shape), const(wse1b.shape),
            const(wse2a.shape), const(wse2b.shape),
            const(w2a.shape), const(w2b.shape), const(b2.shape),
        ],
        out_specs=pl.BlockSpec((NB, oup, Mo), lambda b: (b, 0, 0)),
        compiler_params=pltpu.CompilerParams(
            dimension_semantics=("arbitrary",),
            vmem_limit_bytes=64 * 1024 * 1024),
    )(x3, w1f, b1, wd1f, bd1, wd2f, bd2,
      wse1a, wse1b, wse2a, wse2b, w2a, w2b, b2)
    return out.reshape(B, oup, H, W).astype(f32)
